# baseline (speedup 1.0000x reference)
import functools

import jax
import jax.numpy as jnp
from jax import lax
from jax.experimental import pallas as pl
from jax.experimental.pallas import tpu as pltpu
from jax.experimental.pallas import tpu_sc as plsc

_N = 10000
_E = 320000
_D = 128

_NC = 2
_NS = 16
_NW = _NC * _NS
_CH = 128
_RT = 80
_EP = _NW * _RT * _CH
_SINK = 10000
_NPAD = 10240
_RPT = _NPAD // _NS
_ZR = 128

_mesh = plsc.VectorSubcoreMesh(core_axis_name="c", subcore_axis_name="s")


def _zero_fill(ref, rows, width):
    z16 = jnp.zeros((16,), jnp.float32)

    def body(i, _):
        for j in range(width // 16):
            ref[i, pl.ds(16 * j, 16)] = z16
        return 0

    lax.fori_loop(0, rows, body, 0)


def _count_body(dst_hbm, cnt_hbm, ones_v, idx_v, zbuf, cnt_sh, sem):
    c = lax.axis_index("c")
    s = lax.axis_index("s")
    g = c * _NS + s

    one16 = jnp.ones((16,), jnp.float32)

    def fill(i, _):
        for j in range(_D // 16):
            ones_v[i, pl.ds(16 * j, 16)] = one16
        return 0

    lax.fori_loop(0, _CH, fill, 0)
    _zero_fill(zbuf, _ZR, _D)

    pltpu.sync_copy(dst_hbm.at[pl.ds(g * _RT, _RT)], idx_v)
    for j in range(_RPT // _ZR):
        pltpu.sync_copy(zbuf, cnt_sh.at[pl.ds(s * _RPT + j * _ZR, _ZR)])
    plsc.subcore_barrier()

    _W = 8

    def wave(w, _):
        for k in range(_W):
            pltpu.async_copy(ones_v, cnt_sh.at[idx_v.at[w * _W + k]], sem,
                             add=True)
        for k in range(_W):
            pltpu.make_async_copy(ones_v, cnt_sh.at[idx_v.at[w * _W + k]],
                                  sem).wait()
        return 0

    lax.fori_loop(0, _RT // _W, wave, 0)
    plsc.subcore_barrier()

    for j in range(_RPT // _ZR):
        r0 = s * _RPT + j * _ZR
        pltpu.sync_copy(cnt_sh.at[pl.ds(r0, _ZR)], cnt_hbm.at[c, pl.ds(r0, _ZR)])


_sc_count = functools.partial(
    pl.kernel,
    out_type=jax.ShapeDtypeStruct((_NC, _NPAD, _D), jnp.float32),
    mesh=_mesh,
    scratch_types=[
        pltpu.VMEM((_CH, _D), jnp.float32),
        pltpu.VMEM((_RT, _CH), jnp.int32),
        pltpu.VMEM((_ZR, _D), jnp.float32),
        pltpu.VMEM_SHARED((_NPAD, _D), jnp.float32),
        pltpu.SemaphoreType.DMA,
    ],
)(_count_body)


def _prop_body(hs_hbm, src_hbm, dst_hbm, out_hbm,
               isb0, isb1, idx_d, rows0, rows1, acc_sh, sem0, sem1):
    c = lax.axis_index("c")
    s = lax.axis_index("s")
    g = c * _NS + s

    _zero_fill(rows0, _CH, _D)
    pltpu.sync_copy(dst_hbm.at[pl.ds(g * _RT, _RT)], idx_d)
    for j in range(_RPT // _ZR):
        pltpu.sync_copy(rows0, acc_sh.at[pl.ds(s * _RPT + j * _ZR, _ZR)])
    plsc.subcore_barrier()

    bufs = ((rows0, sem0, isb0), (rows1, sem1, isb1))

    pltpu.sync_copy(src_hbm.at[g * _RT], isb0)
    pltpu.async_copy(hs_hbm.at[isb0], rows0, sem0)

    def step(i, cur, nxt):
        rows_c, sem_c, _ = bufs[cur]
        rows_n, sem_n, isb_n = bufs[nxt]

        @pl.when(i + 1 < _RT)
        def _():
            pltpu.sync_copy(src_hbm.at[g * _RT + i + 1], isb_n)
            pltpu.async_copy(hs_hbm.at[isb_n], rows_n, sem_n)

        pltpu.make_async_copy(hs_hbm.at[pl.ds(0, _CH)], rows_c, sem_c).wait()
        pltpu.sync_copy(rows_c, acc_sh.at[idx_d.at[i]], add=True)

    def outer(i0, _):
        step(2 * i0, 0, 1)
        step(2 * i0 + 1, 1, 0)
        return 0

    lax.fori_loop(0, _RT // 2, outer, 0)
    plsc.subcore_barrier()

    for j in range(_RPT // _ZR):
        r0 = s * _RPT + j * _ZR
        pltpu.sync_copy(acc_sh.at[pl.ds(r0, _ZR)], out_hbm.at[c, pl.ds(r0, _ZR)])


_sc_prop = functools.partial(
    pl.kernel,
    out_type=jax.ShapeDtypeStruct((_NC, _NPAD, _D), jnp.float32),
    mesh=_mesh,
    scratch_types=[
        pltpu.VMEM((_CH,), jnp.int32),
        pltpu.VMEM((_CH,), jnp.int32),
        pltpu.VMEM((_RT, _CH), jnp.int32),
        pltpu.VMEM((_CH, _D), jnp.float32),
        pltpu.VMEM((_CH, _D), jnp.float32),
        pltpu.VMEM_SHARED((_NPAD, _D), jnp.float32),
        pltpu.SemaphoreType.DMA,
        pltpu.SemaphoreType.DMA,
    ],
)(_prop_body)


_BLK = 2000
_GRID = _N // _BLK


def _tc1_body(cnt_ref, x_ref, w1_ref, hs_ref, dis_ref):
    cnt = cnt_ref[0] + cnt_ref[1]
    deg = jnp.max(cnt, axis=1, keepdims=True) + 1.0
    dis = lax.rsqrt(jnp.maximum(deg, 1.0))
    hs_ref[...] = jnp.dot(x_ref[...], w1_ref[...],
                          preferred_element_type=jnp.float32) * dis
    dis_ref[...] = dis


def _tc1(cnt, x, W1):
    return pl.pallas_call(
        _tc1_body,
        grid=(_GRID,),
        in_specs=[
            pl.BlockSpec((_NC, _BLK, _D), lambda i: (0, i, 0)),
            pl.BlockSpec((_BLK, _D), lambda i: (i, 0)),
            pl.BlockSpec((_D, _D), lambda i: (0, 0)),
        ],
        out_specs=[
            pl.BlockSpec((_BLK, _D), lambda i: (i, 0)),
            pl.BlockSpec((_BLK, 1), lambda i: (i, 0)),
        ],
        out_shape=[
            jax.ShapeDtypeStruct((_N, _D), jnp.float32),
            jax.ShapeDtypeStruct((_N, 1), jnp.float32),
        ],
    )(cnt, x, W1)


def _tc2_body(acc_ref, hs1_ref, dis_ref, b1_ref, w2_ref, hs2_ref):
    dis = dis_ref[...]
    h1 = (acc_ref[0] + acc_ref[1] + hs1_ref[...]) * dis + b1_ref[...]
    h1 = jnp.maximum(h1, 0.0)
    hs2_ref[...] = jnp.dot(h1, w2_ref[...],
                           preferred_element_type=jnp.float32) * dis


def _tc2(acc, hs1, dis, b1, W2):
    return pl.pallas_call(
        _tc2_body,
        grid=(_GRID,),
        in_specs=[
            pl.BlockSpec((_NC, _BLK, _D), lambda i: (0, i, 0)),
            pl.BlockSpec((_BLK, _D), lambda i: (i, 0)),
            pl.BlockSpec((_BLK, 1), lambda i: (i, 0)),
            pl.BlockSpec((1, _D), lambda i: (0, 0)),
            pl.BlockSpec((_D, _D), lambda i: (0, 0)),
        ],
        out_specs=pl.BlockSpec((_BLK, _D), lambda i: (i, 0)),
        out_shape=jax.ShapeDtypeStruct((_N, _D), jnp.float32),
    )(acc, hs1, dis, b1, W2)


def _tc3_body(acc_ref, hs2_ref, dis_ref, b2_ref, z_ref):
    z_ref[...] = ((acc_ref[0] + acc_ref[1] + hs2_ref[...]) * dis_ref[...]
                  + b2_ref[...])


def _tc3(acc, hs2, dis, b2):
    return pl.pallas_call(
        _tc3_body,
        grid=(_GRID,),
        in_specs=[
            pl.BlockSpec((_NC, _BLK, _D), lambda i: (0, i, 0)),
            pl.BlockSpec((_BLK, _D), lambda i: (i, 0)),
            pl.BlockSpec((_BLK, 1), lambda i: (i, 0)),
            pl.BlockSpec((1, _D), lambda i: (0, 0)),
        ],
        out_specs=pl.BlockSpec((_BLK, _D), lambda i: (i, 0)),
        out_shape=jax.ShapeDtypeStruct((_N, _D), jnp.float32),
    )(acc, hs2, dis, b2)


def kernel(x, edge_index, W1, b1, W2, b2):
    src = edge_index[0].astype(jnp.int32)
    dst = edge_index[1].astype(jnp.int32)
    pad = _EP - _E
    sink = _SINK + (jnp.arange(pad, dtype=jnp.int32) % (_NPAD - _SINK))
    src2 = jnp.concatenate([src, jnp.zeros((pad,), jnp.int32)]).reshape(-1, _CH)
    dst2 = jnp.concatenate([dst, sink]).reshape(-1, _CH)
    cnt = _sc_count(dst2)
    hs1, dis = _tc1(cnt, x, W1)
    acc1 = _sc_prop(hs1, src2, dst2)
    hs2 = _tc2(acc1, hs1, dis, b1.reshape(1, _D), W2)
    acc2 = _sc_prop(hs2, src2, dst2)
    return _tc3(acc2, hs2, dis, b2.reshape(1, _D))


# speedup vs baseline: 1.3042x; 1.3042x over previous
import functools

import jax
import jax.numpy as jnp
from jax import lax
from jax.experimental import pallas as pl
from jax.experimental.pallas import tpu as pltpu
from jax.experimental.pallas import tpu_sc as plsc

_N = 10000
_E = 320000
_D = 128

_NC = 2
_NS = 16
_NW = _NC * _NS
_CH = 128
_RT = 80
_EP = _NW * _RT * _CH
_SINK = 10000
_EPT = _E // _NW
_ECH = 80
_NCHUNK = _EPT // _ECH
_NPAD = 10240
_RPT = _NPAD // _NS
_ZR = 128

_mesh = plsc.VectorSubcoreMesh(core_axis_name="c", subcore_axis_name="s")


def _zero_fill(ref, rows, width):
    z16 = jnp.zeros((16,), jnp.float32)

    def body(i, _):
        for j in range(width // 16):
            ref[i, pl.ds(16 * j, 16)] = z16
        return 0

    lax.fori_loop(0, rows, body, 0)


def _count_body(dst_hbm, cnt_hbm, ones_v, idx_v, zbuf, cnt_sh, sem):
    c = lax.axis_index("c")
    s = lax.axis_index("s")
    g = c * _NS + s

    one16 = jnp.ones((16,), jnp.float32)

    def fill(i, _):
        for j in range(_D // 16):
            ones_v[i, pl.ds(16 * j, 16)] = one16
        return 0

    lax.fori_loop(0, _CH, fill, 0)
    _zero_fill(zbuf, _ZR, _D)

    pltpu.sync_copy(dst_hbm.at[pl.ds(g * _RT, _RT)], idx_v)
    for j in range(_RPT // _ZR):
        pltpu.sync_copy(zbuf, cnt_sh.at[pl.ds(s * _RPT + j * _ZR, _ZR)])
    plsc.subcore_barrier()

    _W = 8

    def wave(w, _):
        for k in range(_W):
            pltpu.async_copy(ones_v, cnt_sh.at[idx_v.at[w * _W + k]], sem,
                             add=True)
        for k in range(_W):
            pltpu.make_async_copy(ones_v, cnt_sh.at[idx_v.at[w * _W + k]],
                                  sem).wait()
        return 0

    lax.fori_loop(0, _RT // _W, wave, 0)
    plsc.subcore_barrier()

    for j in range(_RPT // _ZR):
        r0 = s * _RPT + j * _ZR
        pltpu.sync_copy(cnt_sh.at[pl.ds(r0, _ZR)], cnt_hbm.at[c, pl.ds(r0, _ZR)])


_sc_count = functools.partial(
    pl.kernel,
    out_type=jax.ShapeDtypeStruct((_NC, _NPAD, _D), jnp.float32),
    mesh=_mesh,
    scratch_types=[
        pltpu.VMEM((_CH, _D), jnp.float32),
        pltpu.VMEM((_RT, _CH), jnp.int32),
        pltpu.VMEM((_ZR, _D), jnp.float32),
        pltpu.VMEM_SHARED((_NPAD, _D), jnp.float32),
        pltpu.SemaphoreType.DMA,
    ],
)(_count_body)


def _prop_body(hs_hbm, src_hbm, dst_hbm, out_hbm,
               idx_s, idx_d, rows, zbuf, acc_sh, sem):
    c = lax.axis_index("c")
    s = lax.axis_index("s")
    g = c * _NS + s

    _zero_fill(zbuf, _ZR, _D)
    for j in range(_RPT // _ZR):
        pltpu.sync_copy(zbuf, acc_sh.at[pl.ds(s * _RPT + j * _ZR, _ZR)])
    plsc.subcore_barrier()

    base = g * _EPT

    def chunk(i, _):
        off = base + i * _ECH
        pltpu.sync_copy(src_hbm.at[pl.ds(off, _ECH)], idx_s)
        pltpu.sync_copy(dst_hbm.at[pl.ds(off, _ECH)], idx_d)
        pltpu.async_copy(hs_hbm.at[idx_s], rows, sem).wait()
        pltpu.sync_copy(rows, acc_sh.at[idx_d], add=True)
        return 0

    lax.fori_loop(0, _NCHUNK, chunk, 0)
    plsc.subcore_barrier()

    for j in range(_RPT // _ZR):
        r0 = s * _RPT + j * _ZR
        pltpu.sync_copy(acc_sh.at[pl.ds(r0, _ZR)], out_hbm.at[c, pl.ds(r0, _ZR)])


_sc_prop = functools.partial(
    pl.kernel,
    out_type=jax.ShapeDtypeStruct((_NC, _NPAD, _D), jnp.float32),
    mesh=_mesh,
    scratch_types=[
        pltpu.VMEM((_ECH,), jnp.int32),
        pltpu.VMEM((_ECH,), jnp.int32),
        pltpu.VMEM((_ECH, _D), jnp.float32),
        pltpu.VMEM((_ZR, _D), jnp.float32),
        pltpu.VMEM_SHARED((_NPAD, _D), jnp.float32),
        pltpu.SemaphoreType.DMA,
    ],
)(_prop_body)


_BLK = 2000
_GRID = _N // _BLK


def _tc1_body(cnt_ref, x_ref, w1_ref, hs_ref, dis_ref):
    cnt = cnt_ref[0] + cnt_ref[1]
    deg = jnp.max(cnt, axis=1, keepdims=True) + 1.0
    dis = lax.rsqrt(jnp.maximum(deg, 1.0))
    hs_ref[...] = jnp.dot(x_ref[...], w1_ref[...],
                          preferred_element_type=jnp.float32) * dis
    dis_ref[...] = dis


def _tc1(cnt, x, W1):
    return pl.pallas_call(
        _tc1_body,
        grid=(_GRID,),
        in_specs=[
            pl.BlockSpec((_NC, _BLK, _D), lambda i: (0, i, 0)),
            pl.BlockSpec((_BLK, _D), lambda i: (i, 0)),
            pl.BlockSpec((_D, _D), lambda i: (0, 0)),
        ],
        out_specs=[
            pl.BlockSpec((_BLK, _D), lambda i: (i, 0)),
            pl.BlockSpec((_BLK, 1), lambda i: (i, 0)),
        ],
        out_shape=[
            jax.ShapeDtypeStruct((_N, _D), jnp.float32),
            jax.ShapeDtypeStruct((_N, 1), jnp.float32),
        ],
    )(cnt, x, W1)


def _tc2_body(acc_ref, hs1_ref, dis_ref, b1_ref, w2_ref, hs2_ref):
    dis = dis_ref[...]
    h1 = (acc_ref[0] + acc_ref[1] + hs1_ref[...]) * dis + b1_ref[...]
    h1 = jnp.maximum(h1, 0.0)
    hs2_ref[...] = jnp.dot(h1, w2_ref[...],
                           preferred_element_type=jnp.float32) * dis


def _tc2(acc, hs1, dis, b1, W2):
    return pl.pallas_call(
        _tc2_body,
        grid=(_GRID,),
        in_specs=[
            pl.BlockSpec((_NC, _BLK, _D), lambda i: (0, i, 0)),
            pl.BlockSpec((_BLK, _D), lambda i: (i, 0)),
            pl.BlockSpec((_BLK, 1), lambda i: (i, 0)),
            pl.BlockSpec((1, _D), lambda i: (0, 0)),
            pl.BlockSpec((_D, _D), lambda i: (0, 0)),
        ],
        out_specs=pl.BlockSpec((_BLK, _D), lambda i: (i, 0)),
        out_shape=jax.ShapeDtypeStruct((_N, _D), jnp.float32),
    )(acc, hs1, dis, b1, W2)


def _tc3_body(acc_ref, hs2_ref, dis_ref, b2_ref, z_ref):
    z_ref[...] = ((acc_ref[0] + acc_ref[1] + hs2_ref[...]) * dis_ref[...]
                  + b2_ref[...])


def _tc3(acc, hs2, dis, b2):
    return pl.pallas_call(
        _tc3_body,
        grid=(_GRID,),
        in_specs=[
            pl.BlockSpec((_NC, _BLK, _D), lambda i: (0, i, 0)),
            pl.BlockSpec((_BLK, _D), lambda i: (i, 0)),
            pl.BlockSpec((_BLK, 1), lambda i: (i, 0)),
            pl.BlockSpec((1, _D), lambda i: (0, 0)),
        ],
        out_specs=pl.BlockSpec((_BLK, _D), lambda i: (i, 0)),
        out_shape=jax.ShapeDtypeStruct((_N, _D), jnp.float32),
    )(acc, hs2, dis, b2)


def kernel(x, edge_index, W1, b1, W2, b2):
    src = edge_index[0].astype(jnp.int32)
    dst = edge_index[1].astype(jnp.int32)
    pad = _EP - _E
    sink = _SINK + (jnp.arange(pad, dtype=jnp.int32) % (_NPAD - _SINK))
    dst2 = jnp.concatenate([dst, sink]).reshape(-1, _CH)
    cnt = _sc_count(dst2)
    hs1, dis = _tc1(cnt, x, W1)
    acc1 = _sc_prop(hs1, src, dst)
    hs2 = _tc2(acc1, hs1, dis, b1.reshape(1, _D), W2)
    acc2 = _sc_prop(hs2, src, dst)
    return _tc3(acc2, hs2, dis, b2.reshape(1, _D))


# speedup vs baseline: 2.2172x; 1.7001x over previous
import functools

import jax
import jax.numpy as jnp
from jax import lax
from jax.experimental import pallas as pl
from jax.experimental.pallas import tpu as pltpu
from jax.experimental.pallas import tpu_sc as plsc

_N = 10000
_E = 320000
_D = 128

_NC = 2
_NS = 16
_NW = _NC * _NS
_CH = 128
_RT = 80
_EP = _NW * _RT * _CH
_SINK = 10000
_EPT = _E // _NW
_ECH = 80
_NCHUNK = _EPT // _ECH
_NPAD = 10240
_RPT = _NPAD // _NS
_ZR = 128

_mesh = plsc.VectorSubcoreMesh(core_axis_name="c", subcore_axis_name="s")


def _zero_fill(ref, rows, width):
    z16 = jnp.zeros((16,), jnp.float32)

    def body(i, _):
        for j in range(width // 16):
            ref[i, pl.ds(16 * j, 16)] = z16
        return 0

    lax.fori_loop(0, rows, body, 0)


def _count_body(dst_hbm, cnt_hbm, ones_v, idx_v, zbuf, cnt_sh, sem):
    c = lax.axis_index("c")
    s = lax.axis_index("s")
    g = c * _NS + s

    one16 = jnp.ones((16,), jnp.float32)

    def fill(i, _):
        for j in range(_D // 16):
            ones_v[i, pl.ds(16 * j, 16)] = one16
        return 0

    lax.fori_loop(0, _CH, fill, 0)
    _zero_fill(zbuf, _ZR, _D)

    pltpu.sync_copy(dst_hbm.at[pl.ds(g * _RT, _RT)], idx_v)
    for j in range(_RPT // _ZR):
        pltpu.sync_copy(zbuf, cnt_sh.at[pl.ds(s * _RPT + j * _ZR, _ZR)])
    plsc.subcore_barrier()

    _W = 8

    def wave(w, _):
        for k in range(_W):
            pltpu.async_copy(ones_v, cnt_sh.at[idx_v.at[w * _W + k]], sem,
                             add=True)
        for k in range(_W):
            pltpu.make_async_copy(ones_v, cnt_sh.at[idx_v.at[w * _W + k]],
                                  sem).wait()
        return 0

    lax.fori_loop(0, _RT // _W, wave, 0)
    plsc.subcore_barrier()

    for j in range(_RPT // _ZR):
        r0 = s * _RPT + j * _ZR
        pltpu.sync_copy(cnt_sh.at[pl.ds(r0, _ZR)], cnt_hbm.at[c, pl.ds(r0, _ZR)])


_sc_count = functools.partial(
    pl.kernel,
    out_type=jax.ShapeDtypeStruct((_NC, _NPAD, _D), jnp.float32),
    mesh=_mesh,
    scratch_types=[
        pltpu.VMEM((_CH, _D), jnp.float32),
        pltpu.VMEM((_RT, _CH), jnp.int32),
        pltpu.VMEM((_ZR, _D), jnp.float32),
        pltpu.VMEM_SHARED((_NPAD, _D), jnp.float32),
        pltpu.SemaphoreType.DMA,
    ],
)(_count_body)


def _prop_body(hs_hbm, src_hbm, dst_hbm, out_hbm,
               is0, is1, id0, id1, r0, r1, zbuf, acc_sh, gsem, ssem):
    c = lax.axis_index("c")
    s = lax.axis_index("s")
    g = c * _NS + s

    _zero_fill(zbuf, _ZR, _D)
    for j in range(_RPT // _ZR):
        pltpu.sync_copy(zbuf, acc_sh.at[pl.ds(s * _RPT + j * _ZR, _ZR)])
    plsc.subcore_barrier()

    base = g * _EPT
    isb = (is0, is1)
    idb = (id0, id1)
    rb = (r0, r1)

    pltpu.sync_copy(src_hbm.at[pl.ds(base, _ECH)], is0)
    pltpu.sync_copy(dst_hbm.at[pl.ds(base, _ECH)], id0)

    def step(i, cur, nxt):
        @pl.when(i >= 2)
        def _():
            pltpu.make_async_copy(rb[cur], acc_sh.at[idb[cur]], ssem).wait()

        d = pltpu.async_copy(hs_hbm.at[isb[cur]], rb[cur], gsem)

        @pl.when(i + 1 < _NCHUNK)
        def _():
            off = base + (i + 1) * _ECH
            pltpu.sync_copy(src_hbm.at[pl.ds(off, _ECH)], isb[nxt])
            pltpu.sync_copy(dst_hbm.at[pl.ds(off, _ECH)], idb[nxt])

        d.wait()
        pltpu.async_copy(rb[cur], acc_sh.at[idb[cur]], ssem, add=True)

    def outer(i0, _):
        step(2 * i0, 0, 1)
        step(2 * i0 + 1, 1, 0)
        return 0

    lax.fori_loop(0, _NCHUNK // 2, outer, 0)
    step(_NCHUNK - 1, 0, 1)
    pltpu.make_async_copy(r1, acc_sh.at[id1], ssem).wait()
    pltpu.make_async_copy(r0, acc_sh.at[id0], ssem).wait()
    plsc.subcore_barrier()

    for j in range(_RPT // _ZR):
        q0 = s * _RPT + j * _ZR
        pltpu.sync_copy(acc_sh.at[pl.ds(q0, _ZR)], out_hbm.at[c, pl.ds(q0, _ZR)])


_sc_prop = functools.partial(
    pl.kernel,
    out_type=jax.ShapeDtypeStruct((_NC, _NPAD, _D), jnp.float32),
    mesh=_mesh,
    scratch_types=[
        pltpu.VMEM((_ECH,), jnp.int32),
        pltpu.VMEM((_ECH,), jnp.int32),
        pltpu.VMEM((_ECH,), jnp.int32),
        pltpu.VMEM((_ECH,), jnp.int32),
        pltpu.VMEM((_ECH, _D), jnp.float32),
        pltpu.VMEM((_ECH, _D), jnp.float32),
        pltpu.VMEM((_ZR, _D), jnp.float32),
        pltpu.VMEM_SHARED((_NPAD, _D), jnp.float32),
        pltpu.SemaphoreType.DMA,
        pltpu.SemaphoreType.DMA,
    ],
)(_prop_body)


_BLK = 2000
_GRID = _N // _BLK


def _tc1_body(cnt_ref, x_ref, w1_ref, hs_ref, dis_ref):
    cnt = cnt_ref[0] + cnt_ref[1]
    deg = jnp.max(cnt, axis=1, keepdims=True) + 1.0
    dis = lax.rsqrt(jnp.maximum(deg, 1.0))
    hs_ref[...] = jnp.dot(x_ref[...], w1_ref[...],
                          preferred_element_type=jnp.float32) * dis
    dis_ref[...] = dis


def _tc1(cnt, x, W1):
    return pl.pallas_call(
        _tc1_body,
        grid=(_GRID,),
        in_specs=[
            pl.BlockSpec((_NC, _BLK, _D), lambda i: (0, i, 0)),
            pl.BlockSpec((_BLK, _D), lambda i: (i, 0)),
            pl.BlockSpec((_D, _D), lambda i: (0, 0)),
        ],
        out_specs=[
            pl.BlockSpec((_BLK, _D), lambda i: (i, 0)),
            pl.BlockSpec((_BLK, 1), lambda i: (i, 0)),
        ],
        out_shape=[
            jax.ShapeDtypeStruct((_N, _D), jnp.float32),
            jax.ShapeDtypeStruct((_N, 1), jnp.float32),
        ],
    )(cnt, x, W1)


def _tc2_body(acc_ref, hs1_ref, dis_ref, b1_ref, w2_ref, hs2_ref):
    dis = dis_ref[...]
    h1 = (acc_ref[0] + acc_ref[1] + hs1_ref[...]) * dis + b1_ref[...]
    h1 = jnp.maximum(h1, 0.0)
    hs2_ref[...] = jnp.dot(h1, w2_ref[...],
                           preferred_element_type=jnp.float32) * dis


def _tc2(acc, hs1, dis, b1, W2):
    return pl.pallas_call(
        _tc2_body,
        grid=(_GRID,),
        in_specs=[
            pl.BlockSpec((_NC, _BLK, _D), lambda i: (0, i, 0)),
            pl.BlockSpec((_BLK, _D), lambda i: (i, 0)),
            pl.BlockSpec((_BLK, 1), lambda i: (i, 0)),
            pl.BlockSpec((1, _D), lambda i: (0, 0)),
            pl.BlockSpec((_D, _D), lambda i: (0, 0)),
        ],
        out_specs=pl.BlockSpec((_BLK, _D), lambda i: (i, 0)),
        out_shape=jax.ShapeDtypeStruct((_N, _D), jnp.float32),
    )(acc, hs1, dis, b1, W2)


def _tc3_body(acc_ref, hs2_ref, dis_ref, b2_ref, z_ref):
    z_ref[...] = ((acc_ref[0] + acc_ref[1] + hs2_ref[...]) * dis_ref[...]
                  + b2_ref[...])


def _tc3(acc, hs2, dis, b2):
    return pl.pallas_call(
        _tc3_body,
        grid=(_GRID,),
        in_specs=[
            pl.BlockSpec((_NC, _BLK, _D), lambda i: (0, i, 0)),
            pl.BlockSpec((_BLK, _D), lambda i: (i, 0)),
            pl.BlockSpec((_BLK, 1), lambda i: (i, 0)),
            pl.BlockSpec((1, _D), lambda i: (0, 0)),
        ],
        out_specs=pl.BlockSpec((_BLK, _D), lambda i: (i, 0)),
        out_shape=jax.ShapeDtypeStruct((_N, _D), jnp.float32),
    )(acc, hs2, dis, b2)


def kernel(x, edge_index, W1, b1, W2, b2):
    src = edge_index[0].astype(jnp.int32)
    dst = edge_index[1].astype(jnp.int32)
    pad = _EP - _E
    sink = _SINK + (jnp.arange(pad, dtype=jnp.int32) % (_NPAD - _SINK))
    dst2 = jnp.concatenate([dst, sink]).reshape(-1, _CH)
    cnt = _sc_count(dst2)
    hs1, dis = _tc1(cnt, x, W1)
    acc1 = _sc_prop(hs1, src, dst)
    hs2 = _tc2(acc1, hs1, dis, b1.reshape(1, _D), W2)
    acc2 = _sc_prop(hs2, src, dst)
    return _tc3(acc2, hs2, dis, b2.reshape(1, _D))
